# baseline (speedup 1.0000x reference)
import jax
import jax.numpy as jnp
from jax import lax
from jax.experimental import pallas as pl
from jax.experimental.pallas import tpu as pltpu

NTOK = 16384
D = 4096
SQRT_K = 8
NE = SQRT_K * SQRT_K
TOP_K = 8
BLK = 512


def _router_body(log_tau_ref, x_ref, wct_ref, idxt_ref, gatest_ref,
                 scores_ref):
    s = jnp.dot(
        x_ref[...].astype(jnp.bfloat16),
        wct_ref[...].astype(jnp.bfloat16),
        preferred_element_type=jnp.float32,
    )
    row = lax.broadcasted_iota(jnp.int32, (2 * SQRT_K, NE), 0)
    col = lax.broadcasted_iota(jnp.int32, (2 * SQRT_K, NE), 1)
    exp1 = ((row < SQRT_K) & ((col // SQRT_K) == row)).astype(jnp.float32)
    exp2 = ((row >= SQRT_K) & ((col % SQRT_K) == (row - SQRT_K))).astype(
        jnp.float32
    )
    rep1 = jnp.dot(s, exp1, preferred_element_type=jnp.float32,
                   precision=lax.Precision.HIGHEST)
    tile2 = jnp.dot(s, exp2, preferred_element_type=jnp.float32,
                    precision=lax.Precision.HIGHEST)
    scores_ref[...] = rep1 + tile2

    st = jnp.transpose(s)
    scorest = (
        jnp.dot(exp1.T, st, preferred_element_type=jnp.float32,
                precision=lax.Precision.HIGHEST)
        + jnp.dot(exp2.T, st, preferred_element_type=jnp.float32,
                  precision=lax.Precision.HIGHEST)
    )

    tau = jnp.exp(log_tau_ref[0, 0])
    rowf = lax.broadcasted_iota(jnp.int32, (NE, BLK), 0).astype(jnp.float32)
    row8 = lax.broadcasted_iota(jnp.int32, (TOP_K, BLK), 0)
    work = scorest
    vals8 = jnp.zeros((TOP_K, BLK), jnp.float32)
    idx8 = jnp.zeros((TOP_K, BLK), jnp.float32)
    for k in range(TOP_K):
        m = jnp.max(work, axis=0, keepdims=True)
        pick = jnp.min(jnp.where(work == m, rowf, jnp.float32(NE)), axis=0,
                       keepdims=True)
        vals8 = jnp.where(row8 == k, m, vals8)
        idx8 = jnp.where(row8 == k, pick, idx8)
        work = jnp.where(rowf == pick, -jnp.inf, work)

    mx = jnp.max(vals8, axis=0, keepdims=True)
    ex = jnp.exp((vals8 - mx) / tau)
    gatest_ref[...] = ex / jnp.sum(ex, axis=0, keepdims=True)
    idxt_ref[...] = idx8.astype(jnp.int32)


@jax.jit
def kernel(x, W1, W2, log_tau):
    wct = jnp.concatenate([W1, W2], axis=0).T
    lt = log_tau.reshape(1, 1)
    grid = NTOK // BLK
    idxt, gatest, scores = pl.pallas_call(
        _router_body,
        grid=(grid,),
        in_specs=[
            pl.BlockSpec(memory_space=pltpu.SMEM),
            pl.BlockSpec((BLK, D), lambda i: (i, 0)),
            pl.BlockSpec((D, 2 * SQRT_K), lambda i: (0, 0)),
        ],
        out_specs=[
            pl.BlockSpec((TOP_K, BLK), lambda i: (0, i)),
            pl.BlockSpec((TOP_K, BLK), lambda i: (0, i)),
            pl.BlockSpec((BLK, NE), lambda i: (i, 0)),
        ],
        out_shape=[
            jax.ShapeDtypeStruct((TOP_K, NTOK), jnp.int32),
            jax.ShapeDtypeStruct((TOP_K, NTOK), jnp.float32),
            jax.ShapeDtypeStruct((NTOK, NE), jnp.float32),
        ],
    )(lt, x, wct)
    return idxt.T, gatest.T, scores


# speedup vs baseline: 1.0452x; 1.0452x over previous
import jax
import jax.numpy as jnp
from jax import lax
from jax.experimental import pallas as pl
from jax.experimental.pallas import tpu as pltpu

NTOK = 16384
D = 4096
SQRT_K = 8
NE = SQRT_K * SQRT_K
TOP_K = 8
BLK = 1024


def _router_body(log_tau_ref, x_ref, wct_ref, idxt_ref, gatest_ref,
                 scores_ref):
    s = jnp.dot(
        x_ref[...].astype(jnp.bfloat16),
        wct_ref[...].astype(jnp.bfloat16),
        preferred_element_type=jnp.float32,
    )
    row = lax.broadcasted_iota(jnp.int32, (2 * SQRT_K, NE), 0)
    col = lax.broadcasted_iota(jnp.int32, (2 * SQRT_K, NE), 1)
    exp1 = ((row < SQRT_K) & ((col // SQRT_K) == row)).astype(jnp.float32)
    exp2 = ((row >= SQRT_K) & ((col % SQRT_K) == (row - SQRT_K))).astype(
        jnp.float32
    )
    rep1 = jnp.dot(s, exp1, preferred_element_type=jnp.float32,
                   precision=lax.Precision.HIGHEST)
    tile2 = jnp.dot(s, exp2, preferred_element_type=jnp.float32,
                    precision=lax.Precision.HIGHEST)
    scores_ref[...] = rep1 + tile2

    st = jnp.transpose(s)
    scorest = (
        jnp.dot(exp1.T, st, preferred_element_type=jnp.float32,
                precision=lax.Precision.HIGHEST)
        + jnp.dot(exp2.T, st, preferred_element_type=jnp.float32,
                  precision=lax.Precision.HIGHEST)
    )

    tau = jnp.exp(log_tau_ref[0, 0])
    rowf = lax.broadcasted_iota(jnp.int32, (NE, BLK), 0).astype(jnp.float32)
    row8 = lax.broadcasted_iota(jnp.int32, (TOP_K, BLK), 0)
    work = scorest
    vals8 = jnp.zeros((TOP_K, BLK), jnp.float32)
    idx8 = jnp.zeros((TOP_K, BLK), jnp.float32)
    for k in range(TOP_K):
        m = jnp.max(work, axis=0, keepdims=True)
        pick = jnp.min(jnp.where(work == m, rowf, jnp.float32(NE)), axis=0,
                       keepdims=True)
        vals8 = jnp.where(row8 == k, m, vals8)
        idx8 = jnp.where(row8 == k, pick, idx8)
        work = jnp.where(rowf == pick, -jnp.inf, work)

    mx = jnp.max(vals8, axis=0, keepdims=True)
    ex = jnp.exp((vals8 - mx) / tau)
    gatest_ref[...] = ex / jnp.sum(ex, axis=0, keepdims=True)
    idxt_ref[...] = idx8.astype(jnp.int32)


@jax.jit
def kernel(x, W1, W2, log_tau):
    wct = jnp.concatenate([W1, W2], axis=0).T
    lt = log_tau.reshape(1, 1)
    grid = NTOK // BLK
    idxt, gatest, scores = pl.pallas_call(
        _router_body,
        grid=(grid,),
        in_specs=[
            pl.BlockSpec(memory_space=pltpu.SMEM),
            pl.BlockSpec((BLK, D), lambda i: (i, 0)),
            pl.BlockSpec((D, 2 * SQRT_K), lambda i: (0, 0)),
        ],
        out_specs=[
            pl.BlockSpec((TOP_K, BLK), lambda i: (0, i)),
            pl.BlockSpec((TOP_K, BLK), lambda i: (0, i)),
            pl.BlockSpec((BLK, NE), lambda i: (i, 0)),
        ],
        out_shape=[
            jax.ShapeDtypeStruct((TOP_K, NTOK), jnp.int32),
            jax.ShapeDtypeStruct((TOP_K, NTOK), jnp.float32),
            jax.ShapeDtypeStruct((NTOK, NE), jnp.float32),
        ],
    )(lt, x, wct)
    return idxt.T, gatest.T, scores
